# baseline (speedup 1.0000x reference)
import functools

import jax
import jax.numpy as jnp
from jax.experimental import pallas as pl
from jax.experimental.pallas import tpu as pltpu


def _pick_block(n, target):
    for b in range(min(target, n), 0, -1):
        if n % b == 0 and b % 8 == 0:
            return b
    return n


def _gcn_kernel(x_ref, adj_ref, w1_ref, b1_ref, w2_ref, b2_ref, out_ref,
                s1_ref, s2_ref, s2b_ref, cache_ref, *, bm, mc, nm):
    p = pl.program_id(0)
    m = pl.program_id(1)
    nu = nm - mc

    @pl.when((p == 0) & (m == 0))
    def _():
        s1_ref[:, :] = jnp.dot(x_ref[:, :], w1_ref[:, :],
                               preferred_element_type=jnp.float32)

    @pl.when(p == 0)
    def _():
        agg = jnp.dot(adj_ref[:, :], s1_ref[:, :],
                      preferred_element_type=jnp.float32)
        h = jnp.maximum(agg + b1_ref[0, :], 0.0)
        s2_ref[pl.ds(m * bm, bm), :] = jnp.dot(
            h, w2_ref[:, :], preferred_element_type=jnp.float32)

    @pl.when((p == 0) & (m < mc))
    def _():
        cache_ref[m, :, :] = adj_ref[:, :].astype(jnp.bfloat16)

    @pl.when((p == 1) & (m == 0))
    def _():
        s2b_ref[:, :] = s2_ref[:, :].astype(jnp.bfloat16)

    @pl.when((p == 1) & (m < nu))
    def _():
        out_ref[:, :] = jnp.dot(adj_ref[:, :], s2_ref[:, :],
                                preferred_element_type=jnp.float32) + b2_ref[0, :]

    @pl.when((p == 1) & (m >= nu))
    def _():
        out_ref[:, :] = jnp.dot(cache_ref[m - nu, :, :], s2b_ref[:, :],
                                preferred_element_type=jnp.float32) + b2_ref[0, :]


@jax.jit
def kernel(x, adj, W1, b1, W2, b2):
    n, nfeat = x.shape
    nhid = W1.shape[1]
    nout = W2.shape[1]
    bm = _pick_block(n, 200)
    nm = n // bm
    mc = min(nm - 1, max(0, 0))

    grid = (2, nm)
    body = functools.partial(_gcn_kernel, bm=bm, mc=mc, nm=nm)

    def adj_index(p_, m_):
        return (jnp.where(p_ == 0, m_,
                          jnp.minimum(m_ + mc, nm - 1)), 0)

    def out_index(p_, m_):
        return (jnp.where(p_ == 0, m_,
                          jnp.where(m_ < nm - mc, m_ + mc, m_ - (nm - mc))), 0)

    out = pl.pallas_call(
        body,
        grid=grid,
        in_specs=[
            pl.BlockSpec((n, nfeat), lambda p, m: (0, 0)),
            pl.BlockSpec((bm, n), adj_index),
            pl.BlockSpec((nfeat, nhid), lambda p, m: (0, 0)),
            pl.BlockSpec((1, nhid), lambda p, m: (0, 0)),
            pl.BlockSpec((nhid, nout), lambda p, m: (0, 0)),
            pl.BlockSpec((1, nout), lambda p, m: (0, 0)),
        ],
        out_specs=pl.BlockSpec((bm, nout), out_index),
        out_shape=jax.ShapeDtypeStruct((n, nout), jnp.float32),
        scratch_shapes=[
            pltpu.VMEM((n, nhid), jnp.float32),
            pltpu.VMEM((n, nout), jnp.float32),
            pltpu.VMEM((n, nout), jnp.bfloat16),
            pltpu.VMEM((mc, bm, n), jnp.bfloat16),
        ],
        compiler_params=pltpu.CompilerParams(
            vmem_limit_bytes=64 * 1024 * 1024,
        ),
    )(x, adj, W1, b1.reshape(1, nhid), W2, b2.reshape(1, nout))
    return out


# speedup vs baseline: 1.0002x; 1.0002x over previous
import functools

import jax
import jax.numpy as jnp
from jax.experimental import pallas as pl
from jax.experimental.pallas import tpu as pltpu


def _pick_block(n, target):
    for b in range(min(target, n), 0, -1):
        if n % b == 0 and b % 8 == 0:
            return b
    return n


def _gcn_kernel(x_ref, adj_ref, w1_ref, b1_ref, w2_ref, b2_ref, out_ref,
                s1_ref, s2_ref, s2b_ref, cache_ref, *, bm, mc, nm, sp):
    p = pl.program_id(0)
    m = pl.program_id(1)
    is_cached = (m % sp == 0) & (m >= sp) & (m <= sp * mc)

    @pl.when((p == 0) & (m == 0))
    def _():
        s1_ref[:, :] = jnp.dot(x_ref[:, :], w1_ref[:, :],
                               preferred_element_type=jnp.float32)

    @pl.when(p == 0)
    def _():
        agg = jnp.dot(adj_ref[:, :], s1_ref[:, :],
                      preferred_element_type=jnp.float32)
        h = jnp.maximum(agg + b1_ref[0, :], 0.0)
        s2_ref[pl.ds(m * bm, bm), :] = jnp.dot(
            h, w2_ref[:, :], preferred_element_type=jnp.float32)

    @pl.when((p == 0) & (m < mc))
    def _():
        cache_ref[m, :, :] = adj_ref[:, :].astype(jnp.bfloat16)

    @pl.when((p == 1) & (m == 0))
    def _():
        s2b_ref[:, :] = s2_ref[:, :].astype(jnp.bfloat16)

    @pl.when((p == 1) & jnp.logical_not(is_cached))
    def _():
        out_ref[:, :] = jnp.dot(adj_ref[:, :], s2_ref[:, :],
                                preferred_element_type=jnp.float32) + b2_ref[0, :]

    @pl.when((p == 1) & is_cached)
    def _():
        out_ref[:, :] = jnp.dot(cache_ref[m // sp - 1, :, :], s2b_ref[:, :],
                                preferred_element_type=jnp.float32) + b2_ref[0, :]


@jax.jit
def kernel(x, adj, W1, b1, W2, b2):
    n, nfeat = x.shape
    nhid = W1.shape[1]
    nout = W2.shape[1]
    bm = _pick_block(n, 200)
    nm = n // bm
    mc = min(nm - 1, max(0, (32 * 1024 * 1024) // (bm * n * 2)))
    sp = nm // (mc + 1) if mc else nm

    grid = (2, nm)
    body = functools.partial(_gcn_kernel, bm=bm, mc=mc, nm=nm, sp=sp)

    def adj_index(p_, m_):
        stream = mc + m_ - jnp.minimum(m_ // sp, mc)
        return (jnp.where(p_ == 0, m_, stream), 0)

    def out_index(p_, m_):
        is_c = (m_ % sp == 0) & (m_ >= sp) & (m_ <= sp * mc)
        stream = mc + m_ - jnp.minimum(m_ // sp, mc)
        return (jnp.where(p_ == 0, m_,
                          jnp.where(is_c, m_ // sp - 1, stream)), 0)

    out = pl.pallas_call(
        body,
        grid=grid,
        in_specs=[
            pl.BlockSpec((n, nfeat), lambda p, m: (0, 0)),
            pl.BlockSpec((bm, n), adj_index),
            pl.BlockSpec((nfeat, nhid), lambda p, m: (0, 0)),
            pl.BlockSpec((1, nhid), lambda p, m: (0, 0)),
            pl.BlockSpec((nhid, nout), lambda p, m: (0, 0)),
            pl.BlockSpec((1, nout), lambda p, m: (0, 0)),
        ],
        out_specs=pl.BlockSpec((bm, nout), out_index),
        out_shape=jax.ShapeDtypeStruct((n, nout), jnp.float32),
        scratch_shapes=[
            pltpu.VMEM((n, nhid), jnp.float32),
            pltpu.VMEM((n, nout), jnp.float32),
            pltpu.VMEM((n, nout), jnp.bfloat16),
            pltpu.VMEM((mc, bm, n), jnp.bfloat16),
        ],
        compiler_params=pltpu.CompilerParams(
            vmem_limit_bytes=64 * 1024 * 1024,
        ),
    )(x, adj, W1, b1.reshape(1, nhid), W2, b2.reshape(1, nout))
    return out


# speedup vs baseline: 1.0449x; 1.0447x over previous
import functools

import jax
import jax.numpy as jnp
from jax.experimental import pallas as pl
from jax.experimental.pallas import tpu as pltpu


def _pick_block(n, target):
    for b in range(min(target, n), 0, -1):
        if n % b == 0 and b % 8 == 0:
            return b
    return n


def _gcn_kernel(x_ref, adj_ref, w1_ref, b1_ref, w2_ref, b2_ref, out_ref,
                s1_ref, s2_ref, s2b_ref, cache_ref, *, bm, mc, nm):
    p = pl.program_id(0)
    m = pl.program_id(1)
    nu = nm - mc

    @pl.when((p == 0) & (m == 0))
    def _():
        s1_ref[:, :] = jnp.dot(x_ref[:, :], w1_ref[:, :],
                               preferred_element_type=jnp.float32)

    @pl.when(p == 0)
    def _():
        agg = jnp.dot(adj_ref[:, :], s1_ref[:, :],
                      preferred_element_type=jnp.float32)
        h = jnp.maximum(agg + b1_ref[0, :], 0.0)
        s2_ref[pl.ds(m * bm, bm), :] = jnp.dot(
            h, w2_ref[:, :], preferred_element_type=jnp.float32)

    @pl.when((p == 0) & (m < mc))
    def _():
        cache_ref[m, :, :] = adj_ref[:, :].astype(jnp.bfloat16)

    @pl.when((p == 1) & (m == 0))
    def _():
        s2b_ref[:, :] = s2_ref[:, :].astype(jnp.bfloat16)

    @pl.when((p == 1) & (m < nu))
    def _():
        out_ref[:, :] = jnp.dot(adj_ref[:, :], s2_ref[:, :],
                                preferred_element_type=jnp.float32) + b2_ref[0, :]

    @pl.when((p == 1) & (m >= nu))
    def _():
        out_ref[:, :] = jnp.dot(cache_ref[m - nu, :, :], s2b_ref[:, :],
                                preferred_element_type=jnp.float32) + b2_ref[0, :]


@jax.jit
def kernel(x, adj, W1, b1, W2, b2):
    n, nfeat = x.shape
    nhid = W1.shape[1]
    nout = W2.shape[1]
    bm = _pick_block(n, 400)
    nm = n // bm
    mc = min(nm - 1, max(0, (16 * 1024 * 1024) // (bm * n * 2)))

    grid = (2, nm)
    body = functools.partial(_gcn_kernel, bm=bm, mc=mc, nm=nm)

    def adj_index(p_, m_):
        return (jnp.where(p_ == 0, m_,
                          jnp.minimum(m_ + mc, nm - 1)), 0)

    def out_index(p_, m_):
        return (jnp.where(p_ == 0, m_,
                          jnp.where(m_ < nm - mc, m_ + mc, m_ - (nm - mc))), 0)

    out = pl.pallas_call(
        body,
        grid=grid,
        in_specs=[
            pl.BlockSpec((n, nfeat), lambda p, m: (0, 0)),
            pl.BlockSpec((bm, n), adj_index),
            pl.BlockSpec((nfeat, nhid), lambda p, m: (0, 0)),
            pl.BlockSpec((1, nhid), lambda p, m: (0, 0)),
            pl.BlockSpec((nhid, nout), lambda p, m: (0, 0)),
            pl.BlockSpec((1, nout), lambda p, m: (0, 0)),
        ],
        out_specs=pl.BlockSpec((bm, nout), out_index),
        out_shape=jax.ShapeDtypeStruct((n, nout), jnp.float32),
        scratch_shapes=[
            pltpu.VMEM((n, nhid), jnp.float32),
            pltpu.VMEM((n, nout), jnp.float32),
            pltpu.VMEM((n, nout), jnp.bfloat16),
            pltpu.VMEM((mc, bm, n), jnp.bfloat16),
        ],
        compiler_params=pltpu.CompilerParams(
            vmem_limit_bytes=64 * 1024 * 1024,
        ),
    )(x, adj, W1, b1.reshape(1, nhid), W2, b2.reshape(1, nout))
    return out
